# baseline (speedup 1.0000x reference)
import jax
import jax.numpy as jnp
from jax.experimental import pallas as pl

_BLOCK_COLS = 16384


def _copy_scatter_body(x_ref, o_ref):
    o_ref[...] = x_ref[...]

    @pl.when(pl.program_id(0) == 0)
    def _patch_tile():
        sub = x_ref[0:8, 0:128]
        r = jax.lax.broadcasted_iota(jnp.int32, sub.shape, 0)
        c = jax.lax.broadcasted_iota(jnp.int32, sub.shape, 1)
        row0 = r == 0
        sub = jnp.where(row0 & (c == 0), jnp.float32(88.0), sub)
        sub = jnp.where(row0 & (c == 1), jnp.float32(99.0), sub)
        o_ref[0:8, 0:128] = sub


def kernel(x):
    n, d = x.shape
    xt = x.T
    grid = pl.cdiv(n, _BLOCK_COLS)
    out_t = pl.pallas_call(
        _copy_scatter_body,
        grid=(grid,),
        in_specs=[pl.BlockSpec((d, _BLOCK_COLS), lambda i: (0, i))],
        out_specs=pl.BlockSpec((d, _BLOCK_COLS), lambda i: (0, i)),
        out_shape=jax.ShapeDtypeStruct((d, n), x.dtype),
    )(xt)
    return out_t.T


# speedup vs baseline: 1.0209x; 1.0209x over previous
import jax
import jax.numpy as jnp
from jax.experimental import pallas as pl

_BLOCK_COLS = 32768


def _copy_scatter_body(x_ref, o_ref):
    o_ref[...] = x_ref[...]

    @pl.when(pl.program_id(0) == 0)
    def _patch_tile():
        sub = x_ref[0:8, 0:128]
        r = jax.lax.broadcasted_iota(jnp.int32, sub.shape, 0)
        c = jax.lax.broadcasted_iota(jnp.int32, sub.shape, 1)
        row0 = r == 0
        sub = jnp.where(row0 & (c == 0), jnp.float32(88.0), sub)
        sub = jnp.where(row0 & (c == 1), jnp.float32(99.0), sub)
        o_ref[0:8, 0:128] = sub


def kernel(x):
    n, d = x.shape
    xt = x.T
    grid = pl.cdiv(n, _BLOCK_COLS)
    out_t = pl.pallas_call(
        _copy_scatter_body,
        grid=(grid,),
        in_specs=[pl.BlockSpec((d, _BLOCK_COLS), lambda i: (0, i))],
        out_specs=pl.BlockSpec((d, _BLOCK_COLS), lambda i: (0, i)),
        out_shape=jax.ShapeDtypeStruct((d, n), x.dtype),
    )(xt)
    return out_t.T


# speedup vs baseline: 1.0245x; 1.0035x over previous
import jax
import jax.numpy as jnp
from jax.experimental import pallas as pl

_BLOCK_COLS = 49152


def _copy_scatter_body(x_ref, o_ref):
    o_ref[...] = x_ref[...]

    @pl.when(pl.program_id(0) == 0)
    def _patch_tile():
        sub = x_ref[0:8, 0:128]
        r = jax.lax.broadcasted_iota(jnp.int32, sub.shape, 0)
        c = jax.lax.broadcasted_iota(jnp.int32, sub.shape, 1)
        row0 = r == 0
        sub = jnp.where(row0 & (c == 0), jnp.float32(88.0), sub)
        sub = jnp.where(row0 & (c == 1), jnp.float32(99.0), sub)
        o_ref[0:8, 0:128] = sub


def kernel(x):
    n, d = x.shape
    xt = x.T
    grid = pl.cdiv(n, _BLOCK_COLS)
    out_t = pl.pallas_call(
        _copy_scatter_body,
        grid=(grid,),
        in_specs=[pl.BlockSpec((d, _BLOCK_COLS), lambda i: (0, i))],
        out_specs=pl.BlockSpec((d, _BLOCK_COLS), lambda i: (0, i)),
        out_shape=jax.ShapeDtypeStruct((d, n), x.dtype),
    )(xt)
    return out_t.T
